# baseline (speedup 1.0000x reference)
import jax
import jax.numpy as jnp
from jax import lax
from jax.experimental import pallas as pl
from jax.experimental.pallas import tpu as pltpu

_D = 1024
_SCALE = float(_D) ** 0.5
_SEQ = 2048
_ROWS = 8192
_C = 256
_N = _ROWS // _C
_RING = 4


def _body(seq_hbm, pe_hbm, out_hbm, pe_v, s_v, o_v, pe_sem, s_sems, o_sems):
    def seq_cp(i, slot):
        return pltpu.make_async_copy(
            seq_hbm.at[pl.ds(i * _C, _C), :], s_v.at[slot], s_sems.at[slot]
        )

    def out_cp(i, slot):
        return pltpu.make_async_copy(
            o_v.at[slot], out_hbm.at[pl.ds(i * _C, _C), :], o_sems.at[slot]
        )

    pltpu.make_async_copy(pe_hbm.at[pl.ds(0, _SEQ), :], pe_v, pe_sem).start()
    for k in range(_RING):
        seq_cp(k, k).start()
    pltpu.make_async_copy(pe_hbm.at[pl.ds(0, _SEQ), :], pe_v, pe_sem).wait()

    def step(i, carry):
        slot = lax.rem(i, _RING)
        seq_cp(i, slot).wait()

        @pl.when(i >= _RING)
        def _():
            out_cp(i - _RING, slot).wait()

        pe_base = lax.rem(i * _C, _SEQ)
        o_v[slot] = s_v[slot] * _SCALE + pe_v[pl.ds(pe_base, _C), :]
        out_cp(i, slot).start()

        @pl.when(i + _RING < _N)
        def _():
            seq_cp(i + _RING, slot).start()

        return carry

    lax.fori_loop(0, _N, step, 0)
    for k in range(_RING):
        i = _N - _RING + k
        out_cp(i, i % _RING).wait()


@jax.jit
def _run(sequence, pe):
    batch, seq_len, d = sequence.shape
    rows = batch * seq_len
    seq2d = sequence.reshape(rows, d)
    pe2d = pe.reshape(pe.shape[1], d)
    out = pl.pallas_call(
        _body,
        in_specs=[
            pl.BlockSpec(memory_space=pl.ANY),
            pl.BlockSpec(memory_space=pl.ANY),
        ],
        out_specs=pl.BlockSpec(memory_space=pl.ANY),
        out_shape=jax.ShapeDtypeStruct((rows, d), sequence.dtype),
        scratch_shapes=[
            pltpu.VMEM((_SEQ, d), jnp.float32),
            pltpu.VMEM((_RING, _C, d), jnp.float32),
            pltpu.VMEM((_RING, _C, d), jnp.float32),
            pltpu.SemaphoreType.DMA,
            pltpu.SemaphoreType.DMA((_RING,)),
            pltpu.SemaphoreType.DMA((_RING,)),
        ],
    )(seq2d, pe2d)
    return out.reshape(batch, seq_len, d)


def kernel(sequence, pe, training, mask):
    del training, mask
    return _run(sequence, pe)


# speedup vs baseline: 1.1218x; 1.1218x over previous
import jax
import jax.numpy as jnp
from jax import lax
from jax.experimental import pallas as pl
from jax.experimental.pallas import tpu as pltpu

_D = 1024
_SCALE = float(_D) ** 0.5
_SEQ = 2048
_ROWS = 8192
_C = 1024
_N = _ROWS // _C
_RING = 4


def _body(seq_hbm, pe_hbm, out_hbm, pe_v, s_v, o_v, pe_sem, s_sems, o_sems):
    def seq_cp(i, slot):
        return pltpu.make_async_copy(
            seq_hbm.at[pl.ds(i * _C, _C), :], s_v.at[slot], s_sems.at[slot]
        )

    def out_cp(i, slot):
        return pltpu.make_async_copy(
            o_v.at[slot], out_hbm.at[pl.ds(i * _C, _C), :], o_sems.at[slot]
        )

    pltpu.make_async_copy(pe_hbm.at[pl.ds(0, _SEQ), :], pe_v, pe_sem).start()
    for k in range(_RING):
        seq_cp(k, k).start()
    pltpu.make_async_copy(pe_hbm.at[pl.ds(0, _SEQ), :], pe_v, pe_sem).wait()

    def step(i, carry):
        slot = lax.rem(i, _RING)
        seq_cp(i, slot).wait()

        @pl.when(i >= _RING)
        def _():
            out_cp(i - _RING, slot).wait()

        pe_base = lax.rem(i * _C, _SEQ)
        o_v[slot] = s_v[slot] * _SCALE + pe_v[pl.ds(pe_base, _C), :]
        out_cp(i, slot).start()

        @pl.when(i + _RING < _N)
        def _():
            seq_cp(i + _RING, slot).start()

        return carry

    lax.fori_loop(0, _N, step, 0)
    for k in range(_RING):
        i = _N - _RING + k
        out_cp(i, i % _RING).wait()


@jax.jit
def _run(sequence, pe):
    batch, seq_len, d = sequence.shape
    rows = batch * seq_len
    seq2d = sequence.reshape(rows, d)
    pe2d = pe.reshape(pe.shape[1], d)
    out = pl.pallas_call(
        _body,
        in_specs=[
            pl.BlockSpec(memory_space=pl.ANY),
            pl.BlockSpec(memory_space=pl.ANY),
        ],
        out_specs=pl.BlockSpec(memory_space=pl.ANY),
        out_shape=jax.ShapeDtypeStruct((rows, d), sequence.dtype),
        scratch_shapes=[
            pltpu.VMEM((_SEQ, d), jnp.float32),
            pltpu.VMEM((_RING, _C, d), jnp.float32),
            pltpu.VMEM((_RING, _C, d), jnp.float32),
            pltpu.SemaphoreType.DMA,
            pltpu.SemaphoreType.DMA((_RING,)),
            pltpu.SemaphoreType.DMA((_RING,)),
        ],
    )(seq2d, pe2d)
    return out.reshape(batch, seq_len, d)


def kernel(sequence, pe, training, mask):
    del training, mask
    return _run(sequence, pe)


# speedup vs baseline: 1.1263x; 1.0040x over previous
import jax
import jax.numpy as jnp
from jax import lax
from jax.experimental import pallas as pl
from jax.experimental.pallas import tpu as pltpu

_D = 1024
_SCALE = float(_D) ** 0.5
_SEQ = 2048
_ROWS = 8192
_C = 1024
_N = _ROWS // _C
_RING = 6


def _body(seq_hbm, pe_hbm, out_hbm, pe_v, s_v, o_v, pe_sem, s_sems, o_sems):
    def seq_cp(i, slot):
        return pltpu.make_async_copy(
            seq_hbm.at[pl.ds(i * _C, _C), :], s_v.at[slot], s_sems.at[slot]
        )

    def out_cp(i, slot):
        return pltpu.make_async_copy(
            o_v.at[slot], out_hbm.at[pl.ds(i * _C, _C), :], o_sems.at[slot]
        )

    pltpu.make_async_copy(pe_hbm.at[pl.ds(0, _SEQ), :], pe_v, pe_sem).start()
    for k in range(_RING):
        seq_cp(k, k).start()
    pltpu.make_async_copy(pe_hbm.at[pl.ds(0, _SEQ), :], pe_v, pe_sem).wait()

    def step(i, carry):
        slot = lax.rem(i, _RING)
        seq_cp(i, slot).wait()

        @pl.when(i >= _RING)
        def _():
            out_cp(i - _RING, slot).wait()

        pe_base = lax.rem(i * _C, _SEQ)
        o_v[slot] = s_v[slot] * _SCALE + pe_v[pl.ds(pe_base, _C), :]
        out_cp(i, slot).start()

        @pl.when(i + _RING < _N)
        def _():
            seq_cp(i + _RING, slot).start()

        return carry

    lax.fori_loop(0, _N, step, 0)
    for k in range(_RING):
        i = _N - _RING + k
        out_cp(i, i % _RING).wait()


@jax.jit
def _run(sequence, pe):
    batch, seq_len, d = sequence.shape
    rows = batch * seq_len
    seq2d = sequence.reshape(rows, d)
    pe2d = pe.reshape(pe.shape[1], d)
    out = pl.pallas_call(
        _body,
        in_specs=[
            pl.BlockSpec(memory_space=pl.ANY),
            pl.BlockSpec(memory_space=pl.ANY),
        ],
        out_specs=pl.BlockSpec(memory_space=pl.ANY),
        out_shape=jax.ShapeDtypeStruct((rows, d), sequence.dtype),
        scratch_shapes=[
            pltpu.VMEM((_SEQ, d), jnp.float32),
            pltpu.VMEM((_RING, _C, d), jnp.float32),
            pltpu.VMEM((_RING, _C, d), jnp.float32),
            pltpu.SemaphoreType.DMA,
            pltpu.SemaphoreType.DMA((_RING,)),
            pltpu.SemaphoreType.DMA((_RING,)),
        ],
    )(seq2d, pe2d)
    return out.reshape(batch, seq_len, d)


def kernel(sequence, pe, training, mask):
    del training, mask
    return _run(sequence, pe)
